# baseline (speedup 1.0000x reference)
import functools

import jax
import jax.numpy as jnp
from jax import lax
from jax.experimental import pallas as pl
from jax.experimental.pallas import tpu as pltpu
from jax.experimental.pallas import tpu_sc as plsc

_MR = 0.5


def _tc_stats_body(n_lab, theta_ref, x_ref, ns_ref, tags_ref,
                   colsum_ref, stats_ref, w_ref):
    x = x_ref[:, 0, :]
    theta = theta_ref[0, 0]
    B = x.shape[0]
    rowmax = jnp.max(x, axis=1, keepdims=True)
    rowmin = jnp.min(x, axis=1, keepdims=True)
    ones_row = jnp.ones((8, B), jnp.float32)
    colsum_ref[...] = jax.lax.dot_general(
        ones_row, x, (((1,), (0,)), ((), ())),
        preferred_element_type=jnp.float32,
        precision=jax.lax.Precision.HIGHEST)[0, :]

    nsf = ns_ref[...].astype(jnp.float32)
    s_count = nsf.shape[1]
    filled = jnp.zeros((B, 1), jnp.float32)
    est_acc = jnp.zeros((B, 1), jnp.float32)
    m = rowmax
    for k in range(9):
        if k > 0:
            m = jnp.max(jnp.where(x < m, x, -jnp.inf), axis=1, keepdims=True)
        cnt = jnp.sum((x == m).astype(jnp.float32), axis=1, keepdims=True)
        nmatch = jnp.sum(
            ((nsf >= filled) & (nsf < filled + cnt)).astype(jnp.float32),
            axis=1, keepdims=True)
        est_acc = est_acc + jnp.where(nmatch > 0.0, m, 0.0) * nmatch
        filled = filled + cnt
    est = est_acc * (1.0 / s_count)

    meta_thr = (rowmax - rowmin) * theta + rowmin
    thr = est * (1.0 - _MR) + meta_thr * _MR
    sum_thr = jnp.sum(thr)

    f = x - thr
    sp = jnp.sum(jnp.maximum(f, 0.0) + jnp.log1p(jnp.exp(-jnp.abs(f))))

    tgf = tags_ref[...]
    n_flat = tgf.shape[0]
    slot = lax.iota(jnp.int32, n_flat) % n_lab
    dup = jnp.zeros((n_flat,), jnp.bool_)
    for k in range(1, n_lab):
        dup = dup | ((tgf == jnp.roll(tgf, k)) & (slot >= k))
    w = 1.0 - dup.astype(jnp.float32)
    u_total = jnp.sum(w)
    w_ref[...] = w

    lane = lax.broadcasted_iota(jnp.int32, (1, 128), 1)
    stats_ref[...] = (jnp.where(lane == 0, sp, 0.0)
                      + jnp.where(lane == 1, sum_thr, 0.0)
                      + jnp.where(lane == 2, u_total, 0.0))


@functools.cache
def _make_sc_gather(c_dim, n_idx):
    mesh = plsc.VectorSubcoreMesh(core_axis_name="c", subcore_axis_name="s")

    @functools.partial(
        pl.kernel, mesh=mesh,
        compiler_params=pltpu.CompilerParams(needs_layout_passes=False),
        out_type=jax.ShapeDtypeStruct((16,), jnp.float32),
        scratch_types=[
            pltpu.VMEM((c_dim,), jnp.float32),
            pltpu.VMEM((n_idx,), jnp.int32),
            pltpu.VMEM((n_idx,), jnp.float32),
            pltpu.VMEM((16,), jnp.float32),
        ],
    )
    def sc_gather(colsum_hbm, tags_hbm, w_hbm, out_hbm,
                  table_v, idx_v, w_v, acc_v):
        cid = lax.axis_index("c")
        sid = lax.axis_index("s")

        @pl.when(jnp.logical_and(cid == 0, sid == 0))
        def _():
            pltpu.sync_copy(colsum_hbm, table_v)
            pltpu.sync_copy(tags_hbm, idx_v)
            pltpu.sync_copy(w_hbm, w_v)
            acc = jnp.zeros((16,), jnp.float32)
            for i in range(n_idx // 16):
                idx = idx_v[pl.ds(i * 16, 16)]
                vals = plsc.load_gather(table_v, [idx])
                acc = acc + vals * w_v[pl.ds(i * 16, 16)]
            tot = jnp.sum(acc)
            acc_v[...] = jnp.zeros((16,), jnp.float32) + tot
            pltpu.sync_copy(acc_v, out_hbm)

    return sc_gather


def kernel(logits, mask, tags, threshold, num_stats):
    B, _, C = logits.shape
    n_lab = tags.shape[1]
    theta = threshold.reshape(1, 1)
    tagsf = tags.reshape(B * n_lab)

    colsum, stats, w = pl.pallas_call(
        functools.partial(_tc_stats_body, n_lab),
        out_shape=[
            jax.ShapeDtypeStruct((C,), jnp.float32),
            jax.ShapeDtypeStruct((1, 128), jnp.float32),
            jax.ShapeDtypeStruct((B * n_lab,), jnp.float32),
        ],
    )(theta, logits, num_stats, tagsf)

    g16 = _make_sc_gather(C, B * n_lab)(colsum, tagsf, w)

    sp = stats[0, 0]
    sum_thr = stats[0, 1]
    u_total = stats[0, 2]
    loss = (B * sp + sum_thr * u_total - g16[0]) / (B * B * C)
    return loss


# speedup vs baseline: 1.0139x; 1.0139x over previous
import functools

import jax
import jax.numpy as jnp
from jax import lax
from jax.experimental import pallas as pl
from jax.experimental.pallas import tpu as pltpu
from jax.experimental.pallas import tpu_sc as plsc

_MR = 0.5


def _tc0_body(n_lab, n_chunks, x_ref, tags_ref, colsum_ref, w_ref):
    j = pl.program_id(0)
    xb = x_ref[...]
    ones_row = jnp.ones((8, xb.shape[0]), jnp.float32)
    colsum_ref[...] = jax.lax.dot_general(
        ones_row, xb, (((1,), (0,)), ((), ())),
        preferred_element_type=jnp.float32,
        precision=jax.lax.Precision.HIGHEST)[0, :]

    @pl.when(j == 0)
    def _():
        tgf = tags_ref[...]
        slot = lax.iota(jnp.int32, tgf.shape[0]) % n_lab
        dup = jnp.zeros(tgf.shape, jnp.bool_)
        for k in range(1, n_lab):
            dup = dup | ((tgf == jnp.roll(tgf, k)) & (slot >= k))
        w_ref[...] = 1.0 - dup.astype(jnp.float32)


def _tc1_body(theta_ref, ns_ref, x_hbm, stats_ref, x_vmem, sem):
    pltpu.async_copy(x_hbm, x_vmem, sem).wait()
    x = x_vmem[...]
    theta = theta_ref[0, 0]
    B = x.shape[0]
    rowmax = jnp.max(x, axis=1, keepdims=True)
    rowmin = jnp.min(x, axis=1, keepdims=True)

    nsf = ns_ref[...].astype(jnp.float32)
    s_count = nsf.shape[1]
    filled = jnp.zeros((B, 1), jnp.float32)
    est_acc = jnp.zeros((B, 1), jnp.float32)
    m = rowmax
    for k in range(9):
        if k > 0:
            m = jnp.max(jnp.where(x < m, x, -jnp.inf), axis=1, keepdims=True)
        cnt = jnp.sum((x == m).astype(jnp.float32), axis=1, keepdims=True)
        nmatch = jnp.sum(
            ((nsf >= filled) & (nsf < filled + cnt)).astype(jnp.float32),
            axis=1, keepdims=True)
        est_acc = est_acc + jnp.where(nmatch > 0.0, m, 0.0) * nmatch
        filled = filled + cnt
    est = est_acc * (1.0 / s_count)

    meta_thr = (rowmax - rowmin) * theta + rowmin
    thr = est * (1.0 - _MR) + meta_thr * _MR
    sum_thr = jnp.sum(thr)

    f = x - thr
    sp = jnp.sum(jnp.maximum(f, 0.0) + jnp.log1p(jnp.exp(-jnp.abs(f))))

    lane = lax.broadcasted_iota(jnp.int32, (1, 128), 1)
    stats_ref[...] = (jnp.where(lane == 0, sp, 0.0)
                      + jnp.where(lane == 1, sum_thr, 0.0))


@functools.cache
def _make_sc_gather(c_dim, n_idx):
    mesh = plsc.VectorSubcoreMesh(core_axis_name="c", subcore_axis_name="s")

    @functools.partial(
        pl.kernel, mesh=mesh,
        compiler_params=pltpu.CompilerParams(needs_layout_passes=False),
        out_type=jax.ShapeDtypeStruct((16,), jnp.float32),
        scratch_types=[
            pltpu.VMEM((c_dim,), jnp.float32),
            pltpu.VMEM((n_idx,), jnp.int32),
            pltpu.VMEM((n_idx,), jnp.float32),
            pltpu.VMEM((16,), jnp.float32),
        ],
    )
    def sc_gather(colsum_hbm, tags_hbm, w_hbm, out_hbm,
                  table_v, idx_v, w_v, acc_v):
        cid = lax.axis_index("c")
        sid = lax.axis_index("s")

        @pl.when(jnp.logical_and(cid == 0, sid == 0))
        def _():
            pltpu.sync_copy(colsum_hbm, table_v)
            pltpu.sync_copy(tags_hbm, idx_v)
            pltpu.sync_copy(w_hbm, w_v)
            acc = jnp.zeros((16,), jnp.float32)
            u_acc = jnp.zeros((16,), jnp.float32)
            for i in range(n_idx // 16):
                idx = idx_v[pl.ds(i * 16, 16)]
                vals = plsc.load_gather(table_v, [idx])
                wv = w_v[pl.ds(i * 16, 16)]
                acc = acc + vals * wv
                u_acc = u_acc + wv
            tot = jnp.sum(acc)
            u_tot = jnp.sum(u_acc)
            lane = lax.iota(jnp.int32, 16)
            acc_v[...] = (jnp.where(lane == 0, tot, 0.0)
                          + jnp.where(lane == 1, u_tot, 0.0))
            pltpu.sync_copy(acc_v, out_hbm)

    return sc_gather


def kernel(logits, mask, tags, threshold, num_stats):
    B, _, C = logits.shape
    n_lab = tags.shape[1]
    theta = threshold.reshape(1, 1)
    tagsf = tags.reshape(B * n_lab)
    x2d = logits.reshape(B, C)
    n_chunks = 8
    cb = C // n_chunks

    colsum, w = pl.pallas_call(
        functools.partial(_tc0_body, n_lab, n_chunks),
        grid=(n_chunks,),
        in_specs=[
            pl.BlockSpec((B, cb), lambda j: (0, j)),
            pl.BlockSpec((B * n_lab,), lambda j: (0,)),
        ],
        out_specs=[
            pl.BlockSpec((cb,), lambda j: (j,)),
            pl.BlockSpec((B * n_lab,), lambda j: (0,)),
        ],
        out_shape=[
            jax.ShapeDtypeStruct((C,), jnp.float32),
            jax.ShapeDtypeStruct((B * n_lab,), jnp.float32),
        ],
    )(x2d, tagsf)

    g16 = _make_sc_gather(C, B * n_lab)(colsum, tagsf, w)

    stats = pl.pallas_call(
        _tc1_body,
        in_specs=[
            pl.BlockSpec(memory_space=pltpu.VMEM),
            pl.BlockSpec(memory_space=pltpu.VMEM),
            pl.BlockSpec(memory_space=pl.ANY),
        ],
        out_shape=jax.ShapeDtypeStruct((1, 128), jnp.float32),
        scratch_shapes=[
            pltpu.VMEM((B, C), jnp.float32),
            pltpu.SemaphoreType.DMA,
        ],
    )(theta, num_stats, x2d)

    sp = stats[0, 0]
    sum_thr = stats[0, 1]
    loss = (B * sp + sum_thr * g16[1] - g16[0]) / (B * B * C)
    return loss


# speedup vs baseline: 1.1520x; 1.1362x over previous
import functools

import jax
import jax.numpy as jnp
from jax import lax
from jax.experimental import pallas as pl
from jax.experimental.pallas import tpu as pltpu
from jax.experimental.pallas import tpu_sc as plsc

_MR = 0.5


def _tc0_body(n_lab, x_ref, tags_ref, colsum_ref, w_ref):
    x = x_ref[0]
    colsum_ref[...] = jnp.sum(x, axis=0)

    tgf = tags_ref[...]
    slot = lax.iota(jnp.int32, tgf.shape[0]) % n_lab
    dup = jnp.zeros(tgf.shape, jnp.bool_)
    for k in range(1, n_lab):
        dup = dup | ((tgf == jnp.roll(tgf, k)) & (slot >= k))
    w_ref[...] = 1.0 - dup.astype(jnp.float32)


def _tc1_body(theta_ref, ns_ref, x_ref, stats_ref):
    x = x_ref[0]
    theta = theta_ref[0, 0]
    B = x.shape[0]
    rowmax = jnp.max(x, axis=1, keepdims=True)
    rowmin = jnp.min(x, axis=1, keepdims=True)

    nsf = ns_ref[...].astype(jnp.float32)
    s_count = nsf.shape[1]
    filled = jnp.zeros((B, 1), jnp.float32)
    est_acc = jnp.zeros((B, 1), jnp.float32)
    m = rowmax
    for k in range(9):
        if k > 0:
            m = jnp.max(jnp.where(x < m, x, -jnp.inf), axis=1, keepdims=True)
        cnt = jnp.sum((x == m).astype(jnp.float32), axis=1, keepdims=True)
        nmatch = jnp.sum(
            ((nsf >= filled) & (nsf < filled + cnt)).astype(jnp.float32),
            axis=1, keepdims=True)
        est_acc = est_acc + jnp.where(nmatch > 0.0, m, 0.0) * nmatch
        filled = filled + cnt
    est = est_acc * (1.0 / s_count)

    meta_thr = (rowmax - rowmin) * theta + rowmin
    thr = est * (1.0 - _MR) + meta_thr * _MR
    sum_thr = jnp.sum(thr)

    f = x - thr
    sp = jnp.sum(jnp.maximum(f, 0.0) + jnp.log1p(jnp.exp(-jnp.abs(f))))

    lane = lax.broadcasted_iota(jnp.int32, (1, 128), 1)
    stats_ref[...] = (jnp.where(lane == 0, sp, 0.0)
                      + jnp.where(lane == 1, sum_thr, 0.0))


@functools.cache
def _make_sc_gather(c_dim, n_idx):
    mesh = plsc.VectorSubcoreMesh(
        core_axis_name="c", subcore_axis_name="s", num_cores=1)

    @functools.partial(
        pl.kernel, mesh=mesh,
        compiler_params=pltpu.CompilerParams(needs_layout_passes=False),
        out_type=jax.ShapeDtypeStruct((16,), jnp.float32),
        scratch_types=[
            pltpu.VMEM((c_dim,), jnp.float32),
            pltpu.VMEM((n_idx,), jnp.int32),
            pltpu.VMEM((n_idx,), jnp.float32),
            pltpu.VMEM((16,), jnp.float32),
        ],
    )
    def sc_gather(colsum_hbm, tags_hbm, w_hbm, out_hbm,
                  table_v, idx_v, w_v, acc_v):
        sid = lax.axis_index("s")

        @pl.when(sid == 0)
        def _():
            pltpu.sync_copy(colsum_hbm, table_v)
            pltpu.sync_copy(tags_hbm, idx_v)
            pltpu.sync_copy(w_hbm, w_v)
            acc = jnp.zeros((16,), jnp.float32)
            u_acc = jnp.zeros((16,), jnp.float32)
            for i in range(n_idx // 16):
                idx = idx_v[pl.ds(i * 16, 16)]
                vals = plsc.load_gather(table_v, [idx])
                wv = w_v[pl.ds(i * 16, 16)]
                acc = acc + vals * wv
                u_acc = u_acc + wv
            tot = jnp.sum(acc)
            u_tot = jnp.sum(u_acc)
            lane = lax.iota(jnp.int32, 16)
            acc_v[...] = (jnp.where(lane == 0, tot, 0.0)
                          + jnp.where(lane == 1, u_tot, 0.0))
            pltpu.sync_copy(acc_v, out_hbm)

    return sc_gather


def kernel(logits, mask, tags, threshold, num_stats):
    B, _, C = logits.shape
    n_lab = tags.shape[1]
    theta = threshold.reshape(1, 1)
    tagsf = tags.reshape(B * n_lab)
    x2d = logits.reshape(1, B, C)

    colsum, w = pl.pallas_call(
        functools.partial(_tc0_body, n_lab),
        out_shape=[
            jax.ShapeDtypeStruct((C,), jnp.float32),
            jax.ShapeDtypeStruct((B * n_lab,), jnp.float32),
        ],
    )(x2d, tagsf)

    g16 = _make_sc_gather(C, B * n_lab)(colsum, tagsf, w)

    stats = pl.pallas_call(
        _tc1_body,
        out_shape=jax.ShapeDtypeStruct((1, 128), jnp.float32),
    )(theta, num_stats, x2d)

    sp = stats[0, 0]
    sum_thr = stats[0, 1]
    loss = (B * sp + sum_thr * g16[1] - g16[0]) / (B * B * C)
    return loss


# speedup vs baseline: 1.3820x; 1.1996x over previous
import functools

import jax
import jax.numpy as jnp
from jax import lax
from jax.experimental import pallas as pl
from jax.experimental.pallas import tpu as pltpu
from jax.experimental.pallas import tpu_sc as plsc

_MR = 0.5


def _tc0_body(n_lab, x_hbm, tags_ref, colsum_ref, w_ref, x_vmem, sem):
    pltpu.async_copy(x_hbm, x_vmem, sem).wait()
    x = x_vmem[...]
    colsum_ref[...] = jnp.sum(x, axis=0)

    tgf = tags_ref[...]
    slot = lax.iota(jnp.int32, tgf.shape[0]) % n_lab
    dup = jnp.zeros(tgf.shape, jnp.bool_)
    for k in range(1, n_lab):
        dup = dup | ((tgf == jnp.roll(tgf, k)) & (slot >= k))
    w_ref[...] = 1.0 - dup.astype(jnp.float32)


def _rsum(a):
    return jnp.sum(jnp.sum(a, axis=1, keepdims=True), axis=2, keepdims=True)


def _tc1_body(theta_ref, ns_ref, x_hbm, stats_ref, x_vmem, sem):
    pltpu.async_copy(x_hbm, x_vmem, sem).wait()
    x = x_vmem[...]
    theta = theta_ref[0, 0]
    B = x.shape[0]
    c_total = jnp.float32(x.shape[1] * x.shape[2])
    rowmax = jnp.max(jnp.max(x, axis=1, keepdims=True), axis=2,
                     keepdims=True)
    rowmin = jnp.min(jnp.min(x, axis=1, keepdims=True), axis=2,
                     keepdims=True)

    nsf = ns_ref[...].astype(jnp.float32)
    s_count = nsf.shape[1]
    gt = jnp.zeros((B, 1), jnp.float32)
    est_acc = jnp.zeros((B, 1), jnp.float32)
    m = rowmax
    for k in range(9):
        lt = x < m
        gt_next = (c_total - _rsum(jnp.where(lt, 1.0, 0.0)))[:, :, 0]
        nmatch = jnp.sum(
            ((nsf >= gt) & (nsf < gt_next)).astype(jnp.float32),
            axis=1, keepdims=True)
        est_acc = est_acc + jnp.where(nmatch > 0.0, m[:, :, 0], 0.0) * nmatch
        if k < 8:
            m = jnp.max(jnp.max(jnp.where(lt, x, -jnp.inf), axis=1,
                                keepdims=True), axis=2, keepdims=True)
        gt = gt_next
    est = est_acc * (1.0 / s_count)

    meta_thr = (rowmax - rowmin) * theta + rowmin
    thr = est[:, :, None] * (1.0 - _MR) + meta_thr * _MR
    sum_thr = jnp.sum(thr)

    f = x - thr
    sp = jnp.sum(_rsum(jnp.maximum(f, 0.0)
                       + jnp.log1p(jnp.exp(-jnp.abs(f)))))

    lane = lax.broadcasted_iota(jnp.int32, (1, 128), 1)
    stats_ref[...] = (jnp.where(lane == 0, sp, 0.0)
                      + jnp.where(lane == 1, sum_thr, 0.0))


@functools.cache
def _make_sc_gather(c_dim, n_idx):
    mesh = plsc.VectorSubcoreMesh(
        core_axis_name="c", subcore_axis_name="s", num_cores=1)

    @functools.partial(
        pl.kernel, mesh=mesh,
        compiler_params=pltpu.CompilerParams(
            needs_layout_passes=False, skip_device_barrier=True),
        out_type=jax.ShapeDtypeStruct((16,), jnp.float32),
        scratch_types=[
            pltpu.VMEM((c_dim,), jnp.float32),
            pltpu.VMEM((n_idx,), jnp.int32),
            pltpu.VMEM((n_idx,), jnp.float32),
            pltpu.VMEM((16,), jnp.float32),
        ],
    )
    def sc_gather(colsum_hbm, tags_hbm, w_hbm, out_hbm,
                  table_v, idx_v, w_v, acc_v):
        sid = lax.axis_index("s")

        @pl.when(sid == 0)
        def _():
            pltpu.sync_copy(colsum_hbm, table_v)
            pltpu.sync_copy(tags_hbm, idx_v)
            pltpu.sync_copy(w_hbm, w_v)
            acc = jnp.zeros((16,), jnp.float32)
            u_acc = jnp.zeros((16,), jnp.float32)
            for i in range(n_idx // 16):
                idx = idx_v[pl.ds(i * 16, 16)]
                vals = plsc.load_gather(table_v, [idx])
                wv = w_v[pl.ds(i * 16, 16)]
                acc = acc + vals * wv
                u_acc = u_acc + wv
            tot = jnp.sum(acc)
            u_tot = jnp.sum(u_acc)
            lane = lax.iota(jnp.int32, 16)
            acc_v[...] = (jnp.where(lane == 0, tot, 0.0)
                          + jnp.where(lane == 1, u_tot, 0.0))
            pltpu.sync_copy(acc_v, out_hbm)

    return sc_gather


def kernel(logits, mask, tags, threshold, num_stats):
    B, _, C = logits.shape
    n_lab = tags.shape[1]
    theta = threshold.reshape(1, 1)
    tagsf = tags.reshape(B * n_lab)
    r_dim = C // 128
    x3 = pltpu.with_memory_space_constraint(
        logits.reshape(B, r_dim, 128), pltpu.MemorySpace.HBM)

    colsum, w = pl.pallas_call(
        functools.partial(_tc0_body, n_lab),
        in_specs=[
            pl.BlockSpec(memory_space=pl.ANY),
            pl.BlockSpec(memory_space=pltpu.VMEM),
        ],
        out_shape=[
            jax.ShapeDtypeStruct((r_dim, 128), jnp.float32),
            jax.ShapeDtypeStruct((B * n_lab,), jnp.float32),
        ],
        scratch_shapes=[
            pltpu.VMEM((B, r_dim, 128), jnp.float32),
            pltpu.SemaphoreType.DMA,
        ],
    )(x3, tagsf)

    g16 = _make_sc_gather(C, B * n_lab)(colsum.reshape(C), tagsf, w)

    stats = pl.pallas_call(
        _tc1_body,
        in_specs=[
            pl.BlockSpec(memory_space=pltpu.VMEM),
            pl.BlockSpec(memory_space=pltpu.VMEM),
            pl.BlockSpec(memory_space=pl.ANY),
        ],
        out_shape=jax.ShapeDtypeStruct((1, 128), jnp.float32),
        scratch_shapes=[
            pltpu.VMEM((B, r_dim, 128), jnp.float32),
            pltpu.SemaphoreType.DMA,
        ],
    )(theta, num_stats, x3)

    sp = stats[0, 0]
    sum_thr = stats[0, 1]
    loss = (B * sp + sum_thr * g16[1] - g16[0]) / (B * B * C)
    return loss


# speedup vs baseline: 1.3825x; 1.0004x over previous
import functools

import jax
import jax.numpy as jnp
from jax import lax
from jax.experimental import pallas as pl
from jax.experimental.pallas import tpu as pltpu
from jax.experimental.pallas import tpu_sc as plsc

_MR = 0.5


def _tc0_body(n_lab, x_hbm, tags_ref, colsum_ref, w_ref, x_vmem, sem, sem2):
    B = x_vmem.shape[0]
    h = B // 2
    cp1 = pltpu.async_copy(x_hbm.at[pl.ds(0, h)], x_vmem.at[pl.ds(0, h)], sem)
    cp2 = pltpu.async_copy(x_hbm.at[pl.ds(h, h)], x_vmem.at[pl.ds(h, h)], sem2)
    cp1.wait()
    acc = jnp.sum(x_vmem[pl.ds(0, h)], axis=0)
    cp2.wait()
    colsum_ref[...] = acc + jnp.sum(x_vmem[pl.ds(h, h)], axis=0)

    tgf = tags_ref[...]
    slot = lax.iota(jnp.int32, tgf.shape[0]) % n_lab
    dup = jnp.zeros(tgf.shape, jnp.bool_)
    for k in range(1, n_lab):
        dup = dup | ((tgf == jnp.roll(tgf, k)) & (slot >= k))
    w_ref[...] = 1.0 - dup.astype(jnp.float32)


def _rsum(a):
    return jnp.sum(jnp.sum(a, axis=1, keepdims=True), axis=2, keepdims=True)


def _tc1_body(theta_ref, ns_ref, x_hbm, sp_ref, st_ref, x_vmem, sem):
    pltpu.async_copy(x_hbm, x_vmem, sem).wait()
    x = x_vmem[...]
    theta = theta_ref[0, 0]
    B = x.shape[0]
    c_total = jnp.float32(x.shape[1] * x.shape[2])
    rowmax = jnp.max(jnp.max(x, axis=1, keepdims=True), axis=2,
                     keepdims=True)
    rowmin = jnp.min(jnp.min(x, axis=1, keepdims=True), axis=2,
                     keepdims=True)

    nsf = ns_ref[...].astype(jnp.float32)
    s_count = nsf.shape[1]

    def round_fn(_, carry):
        m, gt, est_acc = carry
        lt = x < m
        gt_next = (c_total - _rsum(jnp.where(lt, 1.0, 0.0)))[:, :, 0]
        nmatch = jnp.sum(
            ((nsf >= gt) & (nsf < gt_next)).astype(jnp.float32),
            axis=1, keepdims=True)
        est_acc = est_acc + jnp.where(nmatch > 0.0, m[:, :, 0], 0.0) * nmatch
        m = jnp.max(jnp.max(jnp.where(lt, x, -jnp.inf), axis=1,
                            keepdims=True), axis=2, keepdims=True)
        return m, gt_next, est_acc

    _, _, est_acc = lax.fori_loop(
        0, 9, round_fn,
        (rowmax, jnp.zeros((B, 1), jnp.float32),
         jnp.zeros((B, 1), jnp.float32)))
    est = est_acc * (1.0 / s_count)

    meta_thr = (rowmax - rowmin) * theta + rowmin
    thr = est[:, :, None] * (1.0 - _MR) + meta_thr * _MR
    sum_thr = jnp.sum(thr)

    f = x - thr
    sp = jnp.sum(_rsum(jnp.maximum(f, 0.0)
                       + jnp.log1p(jnp.exp(-jnp.abs(f)))))

    sp_ref[0, 0] = sp
    st_ref[0, 0] = sum_thr


@functools.cache
def _make_sc_gather(c_dim, n_idx):
    mesh = plsc.VectorSubcoreMesh(
        core_axis_name="c", subcore_axis_name="s", num_cores=1)

    @functools.partial(
        pl.kernel, mesh=mesh,
        compiler_params=pltpu.CompilerParams(
            needs_layout_passes=False, skip_device_barrier=True),
        out_type=jax.ShapeDtypeStruct((16,), jnp.float32),
        scratch_types=[
            pltpu.VMEM((c_dim,), jnp.float32),
            pltpu.VMEM((n_idx,), jnp.int32),
            pltpu.VMEM((n_idx,), jnp.float32),
            pltpu.VMEM((16,), jnp.float32),
        ],
    )
    def sc_gather(colsum_hbm, tags_hbm, w_hbm, out_hbm,
                  table_v, idx_v, w_v, acc_v):
        sid = lax.axis_index("s")

        @pl.when(sid == 0)
        def _():
            pltpu.sync_copy(colsum_hbm, table_v)
            pltpu.sync_copy(tags_hbm, idx_v)
            pltpu.sync_copy(w_hbm, w_v)
            acc = jnp.zeros((16,), jnp.float32)
            u_acc = jnp.zeros((16,), jnp.float32)
            for i in range(n_idx // 16):
                idx = idx_v[pl.ds(i * 16, 16)]
                vals = plsc.load_gather(table_v, [idx])
                wv = w_v[pl.ds(i * 16, 16)]
                acc = acc + vals * wv
                u_acc = u_acc + wv
            tot = jnp.sum(acc)
            u_tot = jnp.sum(u_acc)
            lane = lax.iota(jnp.int32, 16)
            acc_v[...] = (jnp.where(lane == 0, tot, 0.0)
                          + jnp.where(lane == 1, u_tot, 0.0))
            pltpu.sync_copy(acc_v, out_hbm)

    return sc_gather


def kernel(logits, mask, tags, threshold, num_stats):
    B, _, C = logits.shape
    n_lab = tags.shape[1]
    theta = threshold.reshape(1, 1)
    tagsf = tags.reshape(B * n_lab)
    r_dim = C // 128
    x3 = pltpu.with_memory_space_constraint(
        logits.reshape(B, r_dim, 128), pltpu.MemorySpace.HBM)

    colsum, w = pl.pallas_call(
        functools.partial(_tc0_body, n_lab),
        in_specs=[
            pl.BlockSpec(memory_space=pl.ANY),
            pl.BlockSpec(memory_space=pltpu.VMEM),
        ],
        out_shape=[
            jax.ShapeDtypeStruct((r_dim, 128), jnp.float32),
            jax.ShapeDtypeStruct((B * n_lab,), jnp.float32),
        ],
        scratch_shapes=[
            pltpu.VMEM((B, r_dim, 128), jnp.float32),
            pltpu.SemaphoreType.DMA,
            pltpu.SemaphoreType.DMA,
        ],
    )(x3, tagsf)

    g16 = _make_sc_gather(C, B * n_lab)(colsum.reshape(C), tagsf, w)

    sp2, st2 = pl.pallas_call(
        _tc1_body,
        in_specs=[
            pl.BlockSpec(memory_space=pltpu.VMEM),
            pl.BlockSpec(memory_space=pltpu.VMEM),
            pl.BlockSpec(memory_space=pl.ANY),
        ],
        out_shape=[
            jax.ShapeDtypeStruct((1, 1), jnp.float32),
            jax.ShapeDtypeStruct((1, 1), jnp.float32),
        ],
        out_specs=[
            pl.BlockSpec(memory_space=pltpu.SMEM),
            pl.BlockSpec(memory_space=pltpu.SMEM),
        ],
        scratch_shapes=[
            pltpu.VMEM((B, r_dim, 128), jnp.float32),
            pltpu.SemaphoreType.DMA,
        ],
    )(theta, num_stats, x3)

    loss = (B * sp2[0, 0] + st2[0, 0] * g16[1] - g16[0]) / (B * B * C)
    return loss


# speedup vs baseline: 1.4248x; 1.0306x over previous
import functools

import jax
import jax.numpy as jnp
from jax import lax
from jax.experimental import pallas as pl
from jax.experimental.pallas import tpu as pltpu
from jax.experimental.pallas import tpu_sc as plsc

_MR = 0.5


def _tc0_body(n_lab, x_hbm, tags_ref, colsum_ref, w_ref, x_vmem, sem, sem2):
    B = x_vmem.shape[0]
    h = B // 2
    cp1 = pltpu.async_copy(x_hbm.at[pl.ds(0, h)], x_vmem.at[pl.ds(0, h)], sem)
    cp2 = pltpu.async_copy(x_hbm.at[pl.ds(h, h)], x_vmem.at[pl.ds(h, h)], sem2)
    cp1.wait()
    acc = jnp.sum(x_vmem[pl.ds(0, h)], axis=0)
    cp2.wait()
    colsum_ref[...] = acc + jnp.sum(x_vmem[pl.ds(h, h)], axis=0)

    tgf = tags_ref[...]
    slot = lax.iota(jnp.int32, tgf.shape[0]) % n_lab
    dup = jnp.zeros(tgf.shape, jnp.bool_)
    for k in range(1, n_lab):
        dup = dup | ((tgf == jnp.roll(tgf, k)) & (slot >= k))
    w_ref[...] = 1.0 - dup.astype(jnp.float32)


def _rsum(a):
    return jnp.sum(jnp.sum(a, axis=1, keepdims=True), axis=2, keepdims=True)


def _tc1_body(theta_ref, ns_ref, x_hbm, sp_ref, st_ref, x_vmem, sem):
    pltpu.async_copy(x_hbm, x_vmem, sem).wait()
    x = x_vmem[...]
    theta = theta_ref[0, 0]
    B = x.shape[0]
    c_total = jnp.float32(x.shape[1] * x.shape[2])
    rowmax = jnp.max(jnp.max(x, axis=1, keepdims=True), axis=2,
                     keepdims=True)
    rowmin = jnp.min(jnp.min(x, axis=1, keepdims=True), axis=2,
                     keepdims=True)

    nsf = ns_ref[...].astype(jnp.float32)
    s_count = nsf.shape[1]

    gt = jnp.zeros((B, 1), jnp.float32)
    est_acc = jnp.zeros((B, 1), jnp.float32)
    m = rowmax
    for k in range(9):
        lt = x < m
        gt_next = (c_total - _rsum(jnp.where(lt, 1.0, 0.0)))[:, :, 0]
        nmatch = jnp.sum(
            ((nsf >= gt) & (nsf < gt_next)).astype(jnp.float32),
            axis=1, keepdims=True)
        est_acc = est_acc + jnp.where(nmatch > 0.0, m[:, :, 0], 0.0) * nmatch
        if k < 8:
            m = jnp.max(jnp.max(jnp.where(lt, x, -jnp.inf), axis=1,
                                keepdims=True), axis=2, keepdims=True)
        gt = gt_next
    est = est_acc * (1.0 / s_count)

    meta_thr = (rowmax - rowmin) * theta + rowmin
    thr = est[:, :, None] * (1.0 - _MR) + meta_thr * _MR
    sum_thr = jnp.sum(thr)

    f = x - thr
    sp = jnp.sum(_rsum(jnp.maximum(f, 0.0)
                       + jnp.log1p(jnp.exp(-jnp.abs(f)))))

    sp_ref[0, 0] = sp
    st_ref[0, 0] = sum_thr


@functools.cache
def _make_sc_gather(c_dim, n_idx):
    mesh = plsc.VectorSubcoreMesh(
        core_axis_name="c", subcore_axis_name="s", num_cores=1)

    @functools.partial(
        pl.kernel, mesh=mesh,
        compiler_params=pltpu.CompilerParams(
            needs_layout_passes=False, skip_device_barrier=True),
        out_type=jax.ShapeDtypeStruct((16,), jnp.float32),
        scratch_types=[
            pltpu.VMEM((c_dim,), jnp.float32),
            pltpu.VMEM((n_idx,), jnp.int32),
            pltpu.VMEM((n_idx,), jnp.float32),
            pltpu.VMEM((16,), jnp.float32),
        ],
    )
    def sc_gather(colsum_hbm, tags_hbm, w_hbm, out_hbm,
                  table_v, idx_v, w_v, acc_v):
        sid = lax.axis_index("s")

        @pl.when(sid == 0)
        def _():
            pltpu.sync_copy(colsum_hbm, table_v)
            pltpu.sync_copy(tags_hbm, idx_v)
            pltpu.sync_copy(w_hbm, w_v)
            acc = jnp.zeros((16,), jnp.float32)
            u_acc = jnp.zeros((16,), jnp.float32)
            for i in range(n_idx // 16):
                idx = idx_v[pl.ds(i * 16, 16)]
                vals = plsc.load_gather(table_v, [idx])
                wv = w_v[pl.ds(i * 16, 16)]
                acc = acc + vals * wv
                u_acc = u_acc + wv
            tot = jnp.sum(acc)
            u_tot = jnp.sum(u_acc)
            lane = lax.iota(jnp.int32, 16)
            acc_v[...] = (jnp.where(lane == 0, tot, 0.0)
                          + jnp.where(lane == 1, u_tot, 0.0))
            pltpu.sync_copy(acc_v, out_hbm)

    return sc_gather


def kernel(logits, mask, tags, threshold, num_stats):
    B, _, C = logits.shape
    n_lab = tags.shape[1]
    theta = threshold.reshape(1, 1)
    tagsf = tags.reshape(B * n_lab)
    r_dim = C // 128
    x3 = pltpu.with_memory_space_constraint(
        logits.reshape(B, r_dim, 128), pltpu.MemorySpace.HBM)

    colsum, w = pl.pallas_call(
        functools.partial(_tc0_body, n_lab),
        in_specs=[
            pl.BlockSpec(memory_space=pl.ANY),
            pl.BlockSpec(memory_space=pltpu.VMEM),
        ],
        out_shape=[
            jax.ShapeDtypeStruct((r_dim, 128), jnp.float32),
            jax.ShapeDtypeStruct((B * n_lab,), jnp.float32),
        ],
        scratch_shapes=[
            pltpu.VMEM((B, r_dim, 128), jnp.float32),
            pltpu.SemaphoreType.DMA,
            pltpu.SemaphoreType.DMA,
        ],
    )(x3, tagsf)

    g16 = _make_sc_gather(C, B * n_lab)(colsum.reshape(C), tagsf, w)

    sp2, st2 = pl.pallas_call(
        _tc1_body,
        in_specs=[
            pl.BlockSpec(memory_space=pltpu.VMEM),
            pl.BlockSpec(memory_space=pltpu.VMEM),
            pl.BlockSpec(memory_space=pl.ANY),
        ],
        out_shape=[
            jax.ShapeDtypeStruct((1, 1), jnp.float32),
            jax.ShapeDtypeStruct((1, 1), jnp.float32),
        ],
        out_specs=[
            pl.BlockSpec(memory_space=pltpu.SMEM),
            pl.BlockSpec(memory_space=pltpu.SMEM),
        ],
        scratch_shapes=[
            pltpu.VMEM((B, r_dim, 128), jnp.float32),
            pltpu.SemaphoreType.DMA,
        ],
    )(theta, num_stats, x3)

    loss = (B * sp2[0, 0] + st2[0, 0] * g16[1] - g16[0]) / (B * B * C)
    return loss


# speedup vs baseline: 1.4261x; 1.0009x over previous
import functools

import jax
import jax.numpy as jnp
from jax import lax
from jax.experimental import pallas as pl
from jax.experimental.pallas import tpu as pltpu
from jax.experimental.pallas import tpu_sc as plsc

_MR = 0.5


def _tc0_body(n_lab, x_hbm, tags_ref, colsum_ref, w_ref, x_vmem, sem, sem2):
    B = x_vmem.shape[0]
    h = B // 2
    cp1 = pltpu.async_copy(x_hbm.at[pl.ds(0, h)], x_vmem.at[pl.ds(0, h)], sem)
    cp2 = pltpu.async_copy(x_hbm.at[pl.ds(h, h)], x_vmem.at[pl.ds(h, h)], sem2)
    cp1.wait()
    acc = jnp.sum(x_vmem[pl.ds(0, h)], axis=0)
    cp2.wait()
    colsum_ref[...] = acc + jnp.sum(x_vmem[pl.ds(h, h)], axis=0)

    tgf = tags_ref[...]
    slot = lax.iota(jnp.int32, tgf.shape[0]) % n_lab
    dup = jnp.zeros(tgf.shape, jnp.bool_)
    for k in range(1, n_lab):
        dup = dup | ((tgf == jnp.roll(tgf, k)) & (slot >= k))
    w_ref[...] = 1.0 - dup.astype(jnp.float32)


def _rsum(a):
    return jnp.sum(jnp.sum(a, axis=1, keepdims=True), axis=2, keepdims=True)


def _tc1_body(theta_ref, ns_ref, x_hbm, sp_ref, st_ref, x_vmem, sem):
    pltpu.async_copy(x_hbm, x_vmem, sem).wait()
    x = x_vmem[...]
    theta = theta_ref[0, 0]
    B = x.shape[0]
    c_total = jnp.float32(x.shape[1] * x.shape[2])
    rowmax = jnp.max(jnp.max(x, axis=1, keepdims=True), axis=2,
                     keepdims=True)
    rowmin = jnp.min(jnp.min(x, axis=1, keepdims=True), axis=2,
                     keepdims=True)

    nsf = ns_ref[...].astype(jnp.float32)
    s_count = nsf.shape[1]

    gt = jnp.zeros((B, 1), jnp.float32)
    est_acc = jnp.zeros((B, 1), jnp.float32)
    m = rowmax
    for k in range(9):
        lt = x < m
        gt_next = (c_total - _rsum(jnp.where(lt, 1.0, 0.0)))[:, :, 0]
        nmatch = jnp.sum(
            ((nsf >= gt) & (nsf < gt_next)).astype(jnp.float32),
            axis=1, keepdims=True)
        est_acc = est_acc + jnp.where(nmatch > 0.0, m[:, :, 0], 0.0) * nmatch
        if k < 8:
            m = jnp.max(jnp.max(jnp.where(lt, x, -jnp.inf), axis=1,
                                keepdims=True), axis=2, keepdims=True)
        gt = gt_next
    est = est_acc * (1.0 / s_count)

    meta_thr = (rowmax - rowmin) * theta + rowmin
    thr = est[:, :, None] * (1.0 - _MR) + meta_thr * _MR
    sum_thr = jnp.sum(thr)

    f = x - thr
    sp = jnp.sum(_rsum(jnp.maximum(f, 0.0)
                       + jnp.log1p(jnp.exp(-jnp.abs(f)))))

    sp_ref[0, 0] = sp
    st_ref[0, 0] = sum_thr


@functools.cache
def _make_sc_gather(c_dim, n_idx):
    mesh = plsc.VectorSubcoreMesh(
        core_axis_name="c", subcore_axis_name="s", num_cores=1)

    @functools.partial(
        pl.kernel, mesh=mesh,
        compiler_params=pltpu.CompilerParams(needs_layout_passes=False),
        out_type=jax.ShapeDtypeStruct((16,), jnp.float32),
        scratch_types=[
            pltpu.VMEM((c_dim,), jnp.float32),
            pltpu.VMEM((n_idx,), jnp.int32),
            pltpu.VMEM((n_idx,), jnp.float32),
            pltpu.VMEM((16,), jnp.float32),
        ],
    )
    def sc_gather(colsum_hbm, tags_hbm, w_hbm, out_hbm,
                  table_v, idx_v, w_v, acc_v):
        sid = lax.axis_index("s")

        @pl.when(sid == 0)
        def _():
            pltpu.sync_copy(colsum_hbm, table_v)
            pltpu.sync_copy(tags_hbm, idx_v)
            pltpu.sync_copy(w_hbm, w_v)
            acc = jnp.zeros((16,), jnp.float32)
            u_acc = jnp.zeros((16,), jnp.float32)
            for i in range(n_idx // 16):
                idx = idx_v[pl.ds(i * 16, 16)]
                vals = plsc.load_gather(table_v, [idx])
                wv = w_v[pl.ds(i * 16, 16)]
                acc = acc + vals * wv
                u_acc = u_acc + wv
            tot = jnp.sum(acc)
            u_tot = jnp.sum(u_acc)
            lane = lax.iota(jnp.int32, 16)
            acc_v[...] = (jnp.where(lane == 0, tot, 0.0)
                          + jnp.where(lane == 1, u_tot, 0.0))
            pltpu.sync_copy(acc_v, out_hbm)

    return sc_gather


def kernel(logits, mask, tags, threshold, num_stats):
    B, _, C = logits.shape
    n_lab = tags.shape[1]
    theta = threshold.reshape(1, 1)
    tagsf = tags.reshape(B * n_lab)
    r_dim = C // 128
    x3 = pltpu.with_memory_space_constraint(
        logits.reshape(B, r_dim, 128), pltpu.MemorySpace.HBM)

    colsum, w = pl.pallas_call(
        functools.partial(_tc0_body, n_lab),
        in_specs=[
            pl.BlockSpec(memory_space=pl.ANY),
            pl.BlockSpec(memory_space=pltpu.VMEM),
        ],
        out_shape=[
            jax.ShapeDtypeStruct((r_dim, 128), jnp.float32),
            jax.ShapeDtypeStruct((B * n_lab,), jnp.float32),
        ],
        scratch_shapes=[
            pltpu.VMEM((B, r_dim, 128), jnp.float32),
            pltpu.SemaphoreType.DMA,
            pltpu.SemaphoreType.DMA,
        ],
    )(x3, tagsf)

    g16 = _make_sc_gather(C, B * n_lab)(colsum.reshape(C), tagsf, w)

    sp2, st2 = pl.pallas_call(
        _tc1_body,
        in_specs=[
            pl.BlockSpec(memory_space=pltpu.VMEM),
            pl.BlockSpec(memory_space=pltpu.VMEM),
            pl.BlockSpec(memory_space=pl.ANY),
        ],
        out_shape=[
            jax.ShapeDtypeStruct((1, 1), jnp.float32),
            jax.ShapeDtypeStruct((1, 1), jnp.float32),
        ],
        out_specs=[
            pl.BlockSpec(memory_space=pltpu.SMEM),
            pl.BlockSpec(memory_space=pltpu.SMEM),
        ],
        scratch_shapes=[
            pltpu.VMEM((B, r_dim, 128), jnp.float32),
            pltpu.SemaphoreType.DMA,
        ],
    )(theta, num_stats, x3)

    loss = (B * sp2[0, 0] + st2[0, 0] * g16[1] - g16[0]) / (B * B * C)
    return loss
